# Initial kernel scaffold; baseline (speedup 1.0000x reference)
import jax
import jax.numpy as jnp
from jax.experimental import pallas as pl


def kernel(hidden_states, position_ids, Wq, Wk, Wv, Wo, Wkc, Wg_slc, Wg_swa):
    raise NotImplementedError("write your pallas kernel here")


# speedup vs baseline: 1.1258x; 1.1258x over previous
import functools
import math

import jax
import jax.numpy as jnp
from jax.experimental import pallas as pl
from jax.experimental.pallas import tpu as pltpu


def _proj_body(nh, nkv, hd, x_ref, cs_ref, wqkv_ref, wg_ref,
               q_ref, k_ref, v_ref, g_ref):
    x = x_ref[...].astype(jnp.bfloat16)
    qkv = jax.lax.dot_general(x, wqkv_ref[...], (((1,), (0,)), ((), ())),
                              preferred_element_type=jnp.float32)
    gz = jax.lax.dot_general(x, wg_ref[...], (((1,), (0,)), ((), ())),
                             preferred_element_type=jnp.float32)
    g_ref[...] = (jax.nn.sigmoid(gz[:, :nh]) + jax.nn.sigmoid(gz[:, nh:]))

    cos = cs_ref[:, :hd]
    sin = cs_ref[:, hd:]
    half = hd // 2
    scale = 1.0 / math.sqrt(hd)
    for h in range(nh):
        qh = qkv[:, h * hd:(h + 1) * hd]
        qrot = jnp.concatenate([-qh[:, half:], qh[:, :half]], axis=1)
        q_ref[:, h * hd:(h + 1) * hd] = (
            (qh * cos + qrot * sin) * scale).astype(jnp.bfloat16)
    koff = nh * hd
    for h in range(nkv):
        kh = qkv[:, koff + h * hd:koff + (h + 1) * hd]
        krot = jnp.concatenate([-kh[:, half:], kh[:, :half]], axis=1)
        k_ref[:, h * hd:(h + 1) * hd] = (
            kh * cos + krot * sin).astype(jnp.bfloat16)
    voff = (nh + nkv) * hd
    v_ref[...] = qkv[:, voff:].astype(jnp.bfloat16)


def _attn_body(tq, tk, hd, q_ref, k_ref, v_ref, o_ref):
    qi = pl.program_id(2)
    q = q_ref[...]

    def body(kt, carry):
        m, l, acc = carry
        k = k_ref[pl.ds(kt * tk, tk), :]
        s = jax.lax.dot_general(q, k, (((1,), (1,)), ((), ())),
                                preferred_element_type=jnp.float32)
        row = qi * tq + jax.lax.broadcasted_iota(jnp.int32, (tq, tk), 0)
        col = kt * tk + jax.lax.broadcasted_iota(jnp.int32, (tq, tk), 1)
        s = jnp.where(col <= row, s, -jnp.inf)
        m_new = jnp.maximum(m, jnp.max(s, axis=1, keepdims=True))
        p = jnp.exp(s - m_new)
        alpha = jnp.exp(m - m_new)
        l_new = l * alpha + jnp.sum(p, axis=1, keepdims=True)
        v = v_ref[pl.ds(kt * tk, tk), :]
        pv = jax.lax.dot_general(p.astype(jnp.bfloat16), v,
                                 (((1,), (0,)), ((), ())),
                                 preferred_element_type=jnp.float32)
        acc_new = acc * alpha + pv
        return m_new, l_new, acc_new

    nk = (qi * tq) // tk + 1
    m0 = jnp.full((tq, 1), -jnp.inf, dtype=jnp.float32)
    l0 = jnp.zeros((tq, 1), dtype=jnp.float32)
    acc0 = jnp.zeros((tq, hd), dtype=jnp.float32)
    m, l, acc = jax.lax.fori_loop(0, nk, body, (m0, l0, acc0))
    o_ref[...] = (acc / l).astype(jnp.bfloat16)


def _out_body(nh, hd, a_ref, g_ref, wo_ref, o_ref):
    a = a_ref[...].astype(jnp.float32)
    g = g_ref[...]
    cols = [a[:, h * hd:(h + 1) * hd] * g[:, h:h + 1] for h in range(nh)]
    xg = jnp.concatenate(cols, axis=1).astype(jnp.bfloat16)
    o_ref[...] = jax.lax.dot_general(xg, wo_ref[...], (((1,), (0,)), ((), ())),
                                     preferred_element_type=jnp.float32)


def kernel(hidden_states, position_ids, Wq, Wk, Wv, Wo, Wkc, Wg_slc, Wg_swa):
    b, s, dm = hidden_states.shape
    nh = Wg_slc.shape[1]
    hd = 128
    nkv = Wk.shape[1] // hd
    theta = 10000.0
    n = b * s

    x = hidden_states.reshape(n, dm)
    wqkv = jnp.concatenate([Wq, Wk, Wv], axis=1).astype(jnp.bfloat16)
    wg = jnp.concatenate([Wg_slc, Wg_swa], axis=1).astype(jnp.bfloat16)
    wo = Wo.astype(jnp.bfloat16)

    inv_freq = 1.0 / (theta ** (jnp.arange(0, hd, 2, dtype=jnp.float32) / hd))
    freqs = position_ids.reshape(n).astype(jnp.float32)[:, None] * inv_freq[None, :]
    emb = jnp.concatenate([freqs, freqs], axis=1)
    cs = jnp.concatenate([jnp.cos(emb), jnp.sin(emb)], axis=1)

    ts1 = 512
    q, k, v, g = pl.pallas_call(
        functools.partial(_proj_body, nh, nkv, hd),
        grid=(n // ts1,),
        in_specs=[
            pl.BlockSpec((ts1, dm), lambda i: (i, 0)),
            pl.BlockSpec((ts1, 2 * hd), lambda i: (i, 0)),
            pl.BlockSpec((dm, (nh + 2 * nkv) * hd), lambda i: (0, 0)),
            pl.BlockSpec((dm, 2 * nh), lambda i: (0, 0)),
        ],
        out_specs=[
            pl.BlockSpec((ts1, nh * hd), lambda i: (i, 0)),
            pl.BlockSpec((ts1, nkv * hd), lambda i: (i, 0)),
            pl.BlockSpec((ts1, nkv * hd), lambda i: (i, 0)),
            pl.BlockSpec((ts1, nh), lambda i: (i, 0)),
        ],
        out_shape=[
            jax.ShapeDtypeStruct((n, nh * hd), jnp.bfloat16),
            jax.ShapeDtypeStruct((n, nkv * hd), jnp.bfloat16),
            jax.ShapeDtypeStruct((n, nkv * hd), jnp.bfloat16),
            jax.ShapeDtypeStruct((n, nh), jnp.float32),
        ],
    )(x, cs, wqkv, wg)

    tq, tk = 256, 256
    gq = nh // nkv
    attn = pl.pallas_call(
        functools.partial(_attn_body, tq, tk, hd),
        grid=(b, nh, s // tq),
        in_specs=[
            pl.BlockSpec((tq, hd), lambda bi, h, qi: (bi * (s // tq) + qi, h)),
            pl.BlockSpec((s, hd), lambda bi, h, qi: (bi, h // gq)),
            pl.BlockSpec((s, hd), lambda bi, h, qi: (bi, h // gq)),
        ],
        out_specs=pl.BlockSpec((tq, hd), lambda bi, h, qi: (bi * (s // tq) + qi, h)),
        out_shape=jax.ShapeDtypeStruct((n, nh * hd), jnp.bfloat16),
    )(q, k, v)

    ts3 = 512
    out = pl.pallas_call(
        functools.partial(_out_body, nh, hd),
        grid=(n // ts3,),
        in_specs=[
            pl.BlockSpec((ts3, nh * hd), lambda i: (i, 0)),
            pl.BlockSpec((ts3, nh), lambda i: (i, 0)),
            pl.BlockSpec((nh * hd, dm), lambda i: (0, 0)),
        ],
        out_specs=pl.BlockSpec((ts3, dm), lambda i: (i, 0)),
        out_shape=jax.ShapeDtypeStruct((n, dm), jnp.float32),
    )(attn, g, wo)

    return out.reshape(b, s, dm)
